# baseline (speedup 1.0000x reference)
import functools

import jax
import jax.numpy as jnp
from jax import lax
from jax.experimental import pallas as pl
from jax.experimental.pallas import tpu as pltpu
from jax.experimental.pallas import tpu_sc as plsc

N = 10000
D = 128
DFF = 512

NC = 2
NS = 16
NW = NC * NS
L = 16

EB = 128
NPAD = 10240
DEG_T = NPAD // NS
NR = 10112
ROW_T = NR // NS

_MESH = dict(core_axis_name="c", subcore_axis_name="s",
             num_cores=NC, num_subcores=NS)
_SC_PARAMS = pltpu.CompilerParams(needs_layout_passes=False)


def _worker_id():
    return lax.axis_index("c") * NS + lax.axis_index("s")


def _deg_body(nb, src_h, dst_h, w_h, degs_out, degd_out,
              src_v, dst_v, w_v, zb_v, dsem, degs_sp, degd_sp):
    c = lax.axis_index("c")
    s = lax.axis_index("s")
    wid = _worker_id()

    @plsc.parallel_loop(0, DEG_T // L)
    def _(i):
        zb_v[pl.ds(i * L, L)] = jnp.zeros((L,), jnp.float32)

    pltpu.sync_copy(zb_v, degs_sp.at[pl.ds(s * DEG_T, DEG_T)])
    pltpu.sync_copy(zb_v, degd_sp.at[pl.ds(s * DEG_T, DEG_T)])

    pltpu.sync_copy(src_h.at[wid], src_v)
    pltpu.sync_copy(dst_h.at[wid], dst_v)
    pltpu.sync_copy(w_h.at[wid], w_v)
    plsc.subcore_barrier()

    k = 8
    nbatch = nb // k

    def batch(b, carry):
        def fire(i, carry2):
            j = b * k + i
            pltpu.async_copy(w_v.at[j], degs_sp.at[src_v.at[j]], dsem,
                             add=True)
            pltpu.async_copy(w_v.at[j], degd_sp.at[dst_v.at[j]], dsem,
                             add=True)
            return carry2

        lax.fori_loop(0, k, fire, 0)

        def drain(i, carry2):
            j = b * k + i
            pltpu.make_async_copy(w_v.at[j], degs_sp.at[src_v.at[j]],
                                  dsem).wait()
            pltpu.make_async_copy(w_v.at[j], degd_sp.at[dst_v.at[j]],
                                  dsem).wait()
            return carry2

        lax.fori_loop(0, k, drain, 0)
        return carry

    lax.fori_loop(0, nbatch, batch, 0)
    plsc.subcore_barrier()

    sl = pl.ds(s * DEG_T, DEG_T)
    osl = pl.ds(c * NPAD + s * DEG_T, DEG_T)
    pltpu.sync_copy(degs_sp.at[sl], degs_out.at[osl])
    pltpu.sync_copy(degd_sp.at[sl], degd_out.at[osl])


def _deg_call(src2, dst2, w2, nb):
    return pl.kernel(
        functools.partial(_deg_body, nb),
        out_type=(jax.ShapeDtypeStruct((NC * NPAD,), jnp.float32),
                  jax.ShapeDtypeStruct((NC * NPAD,), jnp.float32)),
        mesh=plsc.VectorSubcoreMesh(**_MESH),
        compiler_params=_SC_PARAMS,
        scratch_types=[
            pltpu.VMEM((nb, EB), jnp.int32),
            pltpu.VMEM((nb, EB), jnp.int32),
            pltpu.VMEM((nb, EB), jnp.float32),
            pltpu.VMEM((DEG_T,), jnp.float32),
            pltpu.SemaphoreType.DMA,
            pltpu.VMEM_SHARED((NPAD,), jnp.float32),
            pltpu.VMEM_SHARED((NPAD,), jnp.float32),
        ],
    )(src2, dst2, w2)


BLK = 8
HB = EB // 2


def _edge_body(nb, h_h, src_h, dst_h, w_h, zeros_h, aggp_out,
               srci_v, dsti_v, wf_v, rows_v, isem, gsem, ssem,
               agg_sp):
    c = lax.axis_index("c")
    s = lax.axis_index("s")
    wid = _worker_id()
    nblk = nb // BLK
    bbase = wid * nblk

    rsl = pl.ds(s * ROW_T, ROW_T)
    pltpu.sync_copy(zeros_h.at[rsl], agg_sp.at[rsl])

    def istart(blk, q):
        pltpu.async_copy(src_h.at[bbase + blk], srci_v.at[q], isem)
        pltpu.async_copy(dst_h.at[bbase + blk], dsti_v.at[q], isem)
        pltpu.async_copy(w_h.at[bbase + blk], wf_v.at[q], isem)

    def iwait(blk, q):
        pltpu.make_async_copy(src_h.at[bbase + blk], srci_v.at[q],
                              isem).wait()
        pltpu.make_async_copy(dst_h.at[bbase + blk], dsti_v.at[q],
                              isem).wait()
        pltpu.make_async_copy(w_h.at[bbase + blk], wf_v.at[q],
                              isem).wait()

    def gstart(p, q, off):
        for hf in range(2):
            pltpu.async_copy(h_h.at[srci_v.at[q, off, hf]],
                             rows_v.at[p, pl.ds(hf * HB, HB)],
                             gsem[2 * p + hf])

    def gwait(p, q, off):
        for hf in range(2):
            pltpu.make_async_copy(h_h.at[srci_v.at[q, off, hf]],
                                  rows_v.at[p, pl.ds(hf * HB, HB)],
                                  gsem[2 * p + hf]).wait()

    def sstart(p, q, off):
        pltpu.async_copy(rows_v.at[p], agg_sp.at[dsti_v.at[q, off]],
                         ssem[p], add=True)

    def swait(p, q, off):
        pltpu.make_async_copy(rows_v.at[p], agg_sp.at[dsti_v.at[q, off]],
                              ssem[p]).wait()

    def scale(p, q, off):
        @plsc.parallel_loop(0, EB // L)
        def _(g):
            w16 = wf_v[q, off, pl.ds(g * L, L)]
            for ll in range(L):
                ce = w16[ll]
                e = g * L + ll
                for kk in range(D // L):
                    fsl = pl.ds(kk * L, L)
                    rows_v[p, e, fsl] = rows_v[p, e, fsl] * ce

    plsc.subcore_barrier()

    istart(0, 0)
    iwait(0, 0)
    gstart(0, 0, 0)
    nblk_t = jnp.int32(nblk)

    def step(j, p):
        blk = j // BLK
        off = j % BLK
        q = blk % 2
        gwait(p, q, off)

        @pl.when(j >= 1)
        def _():
            swait(1 - p, ((j - 1) // BLK) % 2, (j - 1) % BLK)

        @pl.when((off == 0) & (blk + 1 < nblk_t))
        def _():
            istart(blk + 1, 1 - q)

        @pl.when((off == BLK - 1) & (blk + 1 < nblk_t))
        def _():
            iwait(blk + 1, 1 - q)

        @pl.when(j + 1 < nb)
        def _():
            jn = j + 1
            gstart(1 - p, (jn // BLK) % 2, jn % BLK)

        scale(p, q, off)
        sstart(p, q, off)

    def body(jj, carry):
        step(2 * jj, 0)
        step(2 * jj + 1, 1)
        return carry

    lax.fori_loop(0, nb // 2, body, 0)
    swait((nb - 1) % 2, ((nb - 1) // BLK) % 2, (nb - 1) % BLK)
    plsc.subcore_barrier()

    pltpu.sync_copy(agg_sp.at[rsl], aggp_out.at[c, rsl])


def _edge_call(h, src4, dst3, w3, zeros, nb):
    return pl.kernel(
        functools.partial(_edge_body, nb),
        out_type=jax.ShapeDtypeStruct((NC, NR, D), jnp.float32),
        mesh=plsc.VectorSubcoreMesh(**_MESH),
        compiler_params=_SC_PARAMS,
        scratch_types=[
            pltpu.VMEM((2, BLK, 2, HB), jnp.int32),
            pltpu.VMEM((2, BLK, EB), jnp.int32),
            pltpu.VMEM((2, BLK, EB), jnp.float32),
            pltpu.VMEM((2, EB, D), jnp.float32),
            pltpu.SemaphoreType.DMA,
            [pltpu.SemaphoreType.DMA] * 4,
            [pltpu.SemaphoreType.DMA] * 2,
            pltpu.VMEM_SHARED((NR, D), jnp.float32),
        ],
    )(h, src4, dst3, w3, zeros)


def _mm_body(x_ref, w_ref, degs_ref, o_ref):
    h = jnp.dot(x_ref[:, :].astype(jnp.bfloat16),
                w_ref[:, :].astype(jnp.bfloat16),
                preferred_element_type=jnp.float32)
    d = degs_ref[0, :N] + degs_ref[1, :N]
    inv = lax.rsqrt(jnp.maximum(d, 1e-12))
    o_ref[:, :] = h * inv[:, None]


def _mm_call(x, w_conv, degs):
    return pl.pallas_call(
        _mm_body,
        out_shape=jax.ShapeDtypeStruct((N, D), jnp.float32),
    )(x, w_conv, degs)


def _tail_body(x_ref, aggp_ref, degd_ref, bconv_ref, gamma_ref, beta_ref,
               w1_ref, b1_ref, w2_ref, b2_ref, o_ref):
    degd = degd_ref[0, :N] + degd_ref[1, :N]
    inv_dst = lax.rsqrt(jnp.maximum(degd, 1e-12))
    gamma = gamma_ref[:][None, :]
    beta = beta_ref[:][None, :]

    agg = ((aggp_ref[0, :N] + aggp_ref[1, :N]) * inv_dst[:, None]
           + bconv_ref[:][None, :])
    t = x_ref[:, :] + agg
    mu = jnp.mean(t, axis=0, keepdims=True)
    var = jnp.mean((t - mu) ** 2, axis=0, keepdims=True)
    x1 = gamma * (t - mu) * lax.rsqrt(var + 1e-5) + beta

    ffh = jnp.maximum(
        jnp.dot(x1.astype(jnp.bfloat16), w1_ref[:, :].astype(jnp.bfloat16),
                preferred_element_type=jnp.float32)
        + b1_ref[:][None, :], 0.0)
    ff = (jnp.dot(ffh.astype(jnp.bfloat16),
                  w2_ref[:, :].astype(jnp.bfloat16),
                  preferred_element_type=jnp.float32)
          + b2_ref[:][None, :])

    t2 = x1 + ff
    mu2 = jnp.mean(t2, axis=0, keepdims=True)
    var2 = jnp.mean((t2 - mu2) ** 2, axis=0, keepdims=True)
    o_ref[:, :] = gamma * (t2 - mu2) * lax.rsqrt(var2 + 1e-5) + beta


def _tail_call(x, aggp, degd, b_conv, gamma, beta, w1, b1, w2, b2):
    return pl.pallas_call(
        _tail_body,
        out_shape=jax.ShapeDtypeStruct((N, D), jnp.float32),
    )(x, aggp, degd, b_conv, gamma, beta, w1, b1, w2, b2)


def kernel(x, edge_index, edge_weight, W_conv, b_conv, gamma, beta,
           W1, b1, W2, b2):
    src = edge_index[0].astype(jnp.int32)
    dst = edge_index[1].astype(jnp.int32)
    w = edge_weight.astype(jnp.float32)
    e = src.shape[0]

    align = NW * EB * 2 * BLK
    e_pad = -(-e // align) * align
    nb = e_pad // (NW * EB)
    pad = e_pad - e

    pad_idx = jnp.arange(pad, dtype=jnp.int32) % N
    src_p = jnp.concatenate([src, pad_idx])
    dst_p = jnp.concatenate([dst, pad_idx])
    w_p = jnp.pad(w, (0, pad))
    src3 = src_p.reshape(NW * nb // BLK, BLK, EB)
    dst3 = dst_p.reshape(NW * nb // BLK, BLK, EB)
    w3 = w_p.reshape(NW * nb // BLK, BLK, EB)
    zeros = jnp.zeros((NR, D), jnp.float32)

    degs, degd = _deg_call(src_p.reshape(NW, nb, EB),
                           dst_p.reshape(NW, nb, EB),
                           w_p.reshape(NW, nb, EB), nb)
    degs = degs.reshape(NC, NPAD)
    degd = degd.reshape(NC, NPAD)
    h = _mm_call(x, W_conv, degs)
    aggp = _edge_call(h, src3.reshape(-1, BLK, 2, HB), dst3, w3, zeros, nb)
    return _tail_call(x, aggp, degd, b_conv, gamma, beta, W1, b1, W2, b2)


# speedup vs baseline: 1.0160x; 1.0160x over previous
import functools

import jax
import jax.numpy as jnp
from jax import lax
from jax.experimental import pallas as pl
from jax.experimental.pallas import tpu as pltpu
from jax.experimental.pallas import tpu_sc as plsc

N = 10000
D = 128
DFF = 512

NC = 2
NS = 16
NW = NC * NS
L = 16

EB = 128
NPAD = 10240
DEG_T = NPAD // NS
NR = 10112
ROW_T = NR // NS

_MESH = dict(core_axis_name="c", subcore_axis_name="s",
             num_cores=NC, num_subcores=NS)
_SC_PARAMS = pltpu.CompilerParams(needs_layout_passes=False)


def _worker_id():
    return lax.axis_index("c") * NS + lax.axis_index("s")


def _deg_body(nb, src_h, dst_h, w_h, degs_out, degd_out,
              src_v, dst_v, w_v, zb_v, dsem, degs_sp, degd_sp):
    c = lax.axis_index("c")
    s = lax.axis_index("s")
    wid = _worker_id()

    @plsc.parallel_loop(0, DEG_T // L)
    def _(i):
        zb_v[pl.ds(i * L, L)] = jnp.zeros((L,), jnp.float32)

    pltpu.sync_copy(zb_v, degs_sp.at[pl.ds(s * DEG_T, DEG_T)])
    pltpu.sync_copy(zb_v, degd_sp.at[pl.ds(s * DEG_T, DEG_T)])

    pltpu.sync_copy(src_h.at[wid], src_v)
    pltpu.sync_copy(dst_h.at[wid], dst_v)
    pltpu.sync_copy(w_h.at[wid], w_v)
    plsc.subcore_barrier()

    k = 8
    nbatch = nb // k

    def batch(b, carry):
        def fire(i, carry2):
            j = b * k + i
            pltpu.async_copy(w_v.at[j], degs_sp.at[src_v.at[j]], dsem,
                             add=True)
            pltpu.async_copy(w_v.at[j], degd_sp.at[dst_v.at[j]], dsem,
                             add=True)
            return carry2

        lax.fori_loop(0, k, fire, 0)

        def drain(i, carry2):
            j = b * k + i
            pltpu.make_async_copy(w_v.at[j], degs_sp.at[src_v.at[j]],
                                  dsem).wait()
            pltpu.make_async_copy(w_v.at[j], degd_sp.at[dst_v.at[j]],
                                  dsem).wait()
            return carry2

        lax.fori_loop(0, k, drain, 0)
        return carry

    lax.fori_loop(0, nbatch, batch, 0)
    plsc.subcore_barrier()

    sl = pl.ds(s * DEG_T, DEG_T)
    osl = pl.ds(c * NPAD + s * DEG_T, DEG_T)
    pltpu.sync_copy(degs_sp.at[sl], degs_out.at[osl])
    pltpu.sync_copy(degd_sp.at[sl], degd_out.at[osl])


def _deg_call(src2, dst2, w2, nb):
    return pl.kernel(
        functools.partial(_deg_body, nb),
        out_type=(jax.ShapeDtypeStruct((NC * NPAD,), jnp.float32),
                  jax.ShapeDtypeStruct((NC * NPAD,), jnp.float32)),
        mesh=plsc.VectorSubcoreMesh(**_MESH),
        compiler_params=_SC_PARAMS,
        scratch_types=[
            pltpu.VMEM((nb, EB), jnp.int32),
            pltpu.VMEM((nb, EB), jnp.int32),
            pltpu.VMEM((nb, EB), jnp.float32),
            pltpu.VMEM((DEG_T,), jnp.float32),
            pltpu.SemaphoreType.DMA,
            pltpu.VMEM_SHARED((NPAD,), jnp.float32),
            pltpu.VMEM_SHARED((NPAD,), jnp.float32),
        ],
    )(src2, dst2, w2)


BLK = 4
HB = EB // 2


def _edge_body(nb, h_h, degs_h, src_h, dst_h, w_h, zeros_h, aggp_out,
               srci_v, dsti_v, wf_v, rows_v, inv_v, isem, gsem, ssem,
               zsem, agg_sp):
    c = lax.axis_index("c")
    s = lax.axis_index("s")
    wid = _worker_id()
    nblk = nb // BLK
    bbase = wid * nblk

    rsl = pl.ds(s * ROW_T, ROW_T)
    pltpu.async_copy(zeros_h.at[rsl], agg_sp.at[rsl], zsem)

    nrow = NPAD // D
    pltpu.sync_copy(degs_h.at[0], inv_v)
    pltpu.sync_copy(degs_h.at[1], rows_v.at[0, pl.ds(0, nrow)])

    @plsc.parallel_loop(0, nrow * (D // L))
    def _(g):
        r = g // (D // L)
        csl = pl.ds((g % (D // L)) * L, L)
        d16 = jnp.maximum(inv_v[r, csl] + rows_v[0, r, csl],
                          jnp.float32(1e-12))
        ibits = plsc.bitcast(d16, jnp.int32)
        y = plsc.bitcast(jnp.int32(0x5F3759DF)
                         - lax.shift_right_logical(ibits, 1), jnp.float32)
        hd = d16 * jnp.float32(0.5)
        for _it in range(3):
            y = y * (jnp.float32(1.5) - hd * y * y)
        inv_v[r, csl] = y

    pltpu.make_async_copy(zeros_h.at[rsl], agg_sp.at[rsl], zsem).wait()

    def istart(blk, q):
        pltpu.async_copy(src_h.at[bbase + blk], srci_v.at[q], isem)
        pltpu.async_copy(dst_h.at[bbase + blk], dsti_v.at[q], isem)
        pltpu.async_copy(w_h.at[bbase + blk], wf_v.at[q], isem)

    def iwait(blk, q):
        pltpu.make_async_copy(src_h.at[bbase + blk], srci_v.at[q],
                              isem).wait()
        pltpu.make_async_copy(dst_h.at[bbase + blk], dsti_v.at[q],
                              isem).wait()
        pltpu.make_async_copy(w_h.at[bbase + blk], wf_v.at[q],
                              isem).wait()

    def gstart(p, q, off):
        for hf in range(2):
            pltpu.async_copy(h_h.at[srci_v.at[q, off, hf]],
                             rows_v.at[p, pl.ds(hf * HB, HB)],
                             gsem[2 * p + hf])

    def gwait(p, q, off):
        for hf in range(2):
            pltpu.make_async_copy(h_h.at[srci_v.at[q, off, hf]],
                                  rows_v.at[p, pl.ds(hf * HB, HB)],
                                  gsem[2 * p + hf]).wait()

    def sstart(p, q, off):
        pltpu.async_copy(rows_v.at[p], agg_sp.at[dsti_v.at[q, off]],
                         ssem[p], add=True)

    def swait(p, q, off):
        pltpu.make_async_copy(rows_v.at[p], agg_sp.at[dsti_v.at[q, off]],
                              ssem[p]).wait()

    def scale(p, q, off):
        @plsc.parallel_loop(0, EB // L)
        def _(g):
            w16 = wf_v[q, off, pl.ds(g * L, L)]
            i16 = srci_v[q, off, g // (HB // L), pl.ds((g % (HB // L)) * L, L)]
            r16 = lax.shift_right_logical(i16, 7)
            c16 = lax.bitwise_and(i16, D - 1)
            inv16 = plsc.load_gather(inv_v, [r16, c16])
            ce16 = w16 * inv16
            for ll in range(L):
                ce = ce16[ll]
                e = g * L + ll
                for kk in range(D // L):
                    fsl = pl.ds(kk * L, L)
                    rows_v[p, e, fsl] = rows_v[p, e, fsl] * ce

    plsc.subcore_barrier()

    istart(0, 0)
    iwait(0, 0)
    gstart(0, 0, 0)
    nblk_t = jnp.int32(nblk)

    def step(j, p):
        blk = j // BLK
        off = j % BLK
        q = blk % 2
        gwait(p, q, off)

        @pl.when(j >= 1)
        def _():
            swait(1 - p, ((j - 1) // BLK) % 2, (j - 1) % BLK)

        @pl.when((off == 0) & (blk + 1 < nblk_t))
        def _():
            istart(blk + 1, 1 - q)

        @pl.when((off == BLK - 1) & (blk + 1 < nblk_t))
        def _():
            iwait(blk + 1, 1 - q)

        @pl.when(j + 1 < nb)
        def _():
            jn = j + 1
            gstart(1 - p, (jn // BLK) % 2, jn % BLK)

        scale(p, q, off)
        sstart(p, q, off)

    def body(jj, carry):
        step(2 * jj, 0)
        step(2 * jj + 1, 1)
        return carry

    lax.fori_loop(0, nb // 2, body, 0)
    swait((nb - 1) % 2, ((nb - 1) // BLK) % 2, (nb - 1) % BLK)
    plsc.subcore_barrier()

    pltpu.sync_copy(agg_sp.at[rsl], aggp_out.at[c, rsl])


def _edge_call(h, degs3, src4, dst3, w3, zeros, nb):
    return pl.kernel(
        functools.partial(_edge_body, nb),
        out_type=jax.ShapeDtypeStruct((NC, NR, D), jnp.float32),
        mesh=plsc.VectorSubcoreMesh(**_MESH),
        compiler_params=_SC_PARAMS,
        scratch_types=[
            pltpu.VMEM((2, BLK, 2, HB), jnp.int32),
            pltpu.VMEM((2, BLK, EB), jnp.int32),
            pltpu.VMEM((2, BLK, EB), jnp.float32),
            pltpu.VMEM((2, EB, D), jnp.float32),
            pltpu.VMEM((NPAD // D, D), jnp.float32),
            pltpu.SemaphoreType.DMA,
            [pltpu.SemaphoreType.DMA] * 4,
            [pltpu.SemaphoreType.DMA] * 2,
            pltpu.SemaphoreType.DMA,
            pltpu.VMEM_SHARED((NR, D), jnp.float32),
        ],
    )(h, degs3, src4, dst3, w3, zeros)


def _mm_body(x_ref, w_ref, o_ref):
    o_ref[:, :] = jnp.dot(x_ref[:, :].astype(jnp.bfloat16),
                          w_ref[:, :].astype(jnp.bfloat16),
                          preferred_element_type=jnp.float32)


def _mm_call(x, w_conv):
    return pl.pallas_call(
        _mm_body,
        out_shape=jax.ShapeDtypeStruct((N, D), jnp.float32),
    )(x, w_conv)


def _tail_body(x_ref, aggp_ref, degd_ref, bconv_ref, gamma_ref, beta_ref,
               w1_ref, b1_ref, w2_ref, b2_ref, o_ref):
    degd = degd_ref[0, :N] + degd_ref[1, :N]
    inv_dst = lax.rsqrt(jnp.maximum(degd, 1e-12))
    gamma = gamma_ref[:][None, :]
    beta = beta_ref[:][None, :]

    agg = ((aggp_ref[0, :N] + aggp_ref[1, :N]) * inv_dst[:, None]
           + bconv_ref[:][None, :])
    t = x_ref[:, :] + agg
    mu = jnp.mean(t, axis=0, keepdims=True)
    var = jnp.mean((t - mu) ** 2, axis=0, keepdims=True)
    x1 = gamma * (t - mu) * lax.rsqrt(var + 1e-5) + beta

    ffh = jnp.maximum(
        jnp.dot(x1.astype(jnp.bfloat16), w1_ref[:, :].astype(jnp.bfloat16),
                preferred_element_type=jnp.float32)
        + b1_ref[:][None, :], 0.0)
    ff = (jnp.dot(ffh.astype(jnp.bfloat16),
                  w2_ref[:, :].astype(jnp.bfloat16),
                  preferred_element_type=jnp.float32)
          + b2_ref[:][None, :])

    t2 = x1 + ff
    mu2 = jnp.mean(t2, axis=0, keepdims=True)
    var2 = jnp.mean((t2 - mu2) ** 2, axis=0, keepdims=True)
    o_ref[:, :] = gamma * (t2 - mu2) * lax.rsqrt(var2 + 1e-5) + beta


def _tail_call(x, aggp, degd, b_conv, gamma, beta, w1, b1, w2, b2):
    return pl.pallas_call(
        _tail_body,
        out_shape=jax.ShapeDtypeStruct((N, D), jnp.float32),
    )(x, aggp, degd, b_conv, gamma, beta, w1, b1, w2, b2)


def kernel(x, edge_index, edge_weight, W_conv, b_conv, gamma, beta,
           W1, b1, W2, b2):
    src = edge_index[0].astype(jnp.int32)
    dst = edge_index[1].astype(jnp.int32)
    w = edge_weight.astype(jnp.float32)
    e = src.shape[0]

    align = NW * EB * 2 * BLK
    e_pad = -(-e // align) * align
    nb = e_pad // (NW * EB)
    pad = e_pad - e

    pad_idx = jnp.arange(pad, dtype=jnp.int32) % N
    src_p = jnp.concatenate([src, pad_idx])
    dst_p = jnp.concatenate([dst, pad_idx])
    w_p = jnp.pad(w, (0, pad))
    src3 = src_p.reshape(NW * nb // BLK, BLK, EB)
    dst3 = dst_p.reshape(NW * nb // BLK, BLK, EB)
    w3 = w_p.reshape(NW * nb // BLK, BLK, EB)
    zeros = jnp.zeros((NR, D), jnp.float32)

    degs, degd = _deg_call(src_p.reshape(NW, nb, EB),
                           dst_p.reshape(NW, nb, EB),
                           w_p.reshape(NW, nb, EB), nb)
    degs3 = degs.reshape(NC, NPAD // D, D)
    degd = degd.reshape(NC, NPAD)
    h = _mm_call(x, W_conv)
    aggp = _edge_call(h, degs3, src3.reshape(-1, BLK, 2, HB), dst3, w3,
                      zeros, nb)
    return _tail_call(x, aggp, degd, b_conv, gamma, beta, W1, b1, W2, b2)


# speedup vs baseline: 1.0203x; 1.0042x over previous
import functools

import jax
import jax.numpy as jnp
from jax import lax
from jax.experimental import pallas as pl
from jax.experimental.pallas import tpu as pltpu
from jax.experimental.pallas import tpu_sc as plsc

N = 10000
D = 128
DFF = 512

NC = 2
NS = 16
NW = NC * NS
L = 16

EB = 128
NPAD = 10240
DEG_T = NPAD // NS
NR = 10112
ROW_T = NR // NS

_MESH = dict(core_axis_name="c", subcore_axis_name="s",
             num_cores=NC, num_subcores=NS)
_SC_PARAMS = pltpu.CompilerParams(needs_layout_passes=False)


def _worker_id():
    return lax.axis_index("c") * NS + lax.axis_index("s")


def _deg_body(nb, src_h, dst_h, w_h, degs_out, degd_out,
              src_v, dst_v, w_v, zb_v, dsem, degs_sp, degd_sp):
    c = lax.axis_index("c")
    s = lax.axis_index("s")
    wid = _worker_id()

    @plsc.parallel_loop(0, DEG_T // L)
    def _(i):
        zb_v[pl.ds(i * L, L)] = jnp.zeros((L,), jnp.float32)

    pltpu.sync_copy(zb_v, degs_sp.at[pl.ds(s * DEG_T, DEG_T)])
    pltpu.sync_copy(zb_v, degd_sp.at[pl.ds(s * DEG_T, DEG_T)])

    pltpu.sync_copy(src_h.at[wid], src_v)
    pltpu.sync_copy(dst_h.at[wid], dst_v)
    pltpu.sync_copy(w_h.at[wid], w_v)
    plsc.subcore_barrier()

    k = 8
    nbatch = nb // k

    def batch(b, carry):
        def fire(i, carry2):
            j = b * k + i
            pltpu.async_copy(w_v.at[j], degs_sp.at[src_v.at[j]], dsem,
                             add=True)
            pltpu.async_copy(w_v.at[j], degd_sp.at[dst_v.at[j]], dsem,
                             add=True)
            return carry2

        lax.fori_loop(0, k, fire, 0)

        def drain(i, carry2):
            j = b * k + i
            pltpu.make_async_copy(w_v.at[j], degs_sp.at[src_v.at[j]],
                                  dsem).wait()
            pltpu.make_async_copy(w_v.at[j], degd_sp.at[dst_v.at[j]],
                                  dsem).wait()
            return carry2

        lax.fori_loop(0, k, drain, 0)
        return carry

    lax.fori_loop(0, nbatch, batch, 0)
    plsc.subcore_barrier()

    sl = pl.ds(s * DEG_T, DEG_T)
    osl = pl.ds(c * NPAD + s * DEG_T, DEG_T)
    pltpu.sync_copy(degs_sp.at[sl], degs_out.at[osl])
    pltpu.sync_copy(degd_sp.at[sl], degd_out.at[osl])


def _deg_call(src2, dst2, w2, nb):
    return pl.kernel(
        functools.partial(_deg_body, nb),
        out_type=(jax.ShapeDtypeStruct((NC * NPAD,), jnp.float32),
                  jax.ShapeDtypeStruct((NC * NPAD,), jnp.float32)),
        mesh=plsc.VectorSubcoreMesh(**_MESH),
        compiler_params=_SC_PARAMS,
        scratch_types=[
            pltpu.VMEM((nb, EB), jnp.int32),
            pltpu.VMEM((nb, EB), jnp.int32),
            pltpu.VMEM((nb, EB), jnp.float32),
            pltpu.VMEM((DEG_T,), jnp.float32),
            pltpu.SemaphoreType.DMA,
            pltpu.VMEM_SHARED((NPAD,), jnp.float32),
            pltpu.VMEM_SHARED((NPAD,), jnp.float32),
        ],
    )(src2, dst2, w2)


BLK = 4
HB = EB // 2


def _edge_body(nb, h_h, degs_h, src_h, dst_h, w_h, zeros_h, aggp_out,
               srci_v, dsti_v, wf_v, rows_v, inv_v, isem, gsem, ssem,
               zsem, agg_sp):
    c = lax.axis_index("c")
    s = lax.axis_index("s")
    wid = _worker_id()
    nblk = nb // BLK
    bbase = wid * nblk

    rsl = pl.ds(s * ROW_T, ROW_T)
    pltpu.async_copy(zeros_h.at[rsl], agg_sp.at[rsl], zsem)

    nrow = NPAD // D
    pltpu.sync_copy(degs_h.at[0], inv_v)
    pltpu.sync_copy(degs_h.at[1], rows_v.at[0, pl.ds(0, nrow)])

    @plsc.parallel_loop(0, nrow * (D // L))
    def _(g):
        r = g // (D // L)
        csl = pl.ds((g % (D // L)) * L, L)
        d16 = jnp.maximum(inv_v[r, csl] + rows_v[0, r, csl],
                          jnp.float32(1e-12))
        ibits = plsc.bitcast(d16, jnp.int32)
        y = plsc.bitcast(jnp.int32(0x5F3759DF)
                         - lax.shift_right_logical(ibits, 1), jnp.float32)
        hd = d16 * jnp.float32(0.5)
        for _it in range(3):
            y = y * (jnp.float32(1.5) - hd * y * y)
        inv_v[r, csl] = y


    def istart(blk, q):
        pltpu.async_copy(src_h.at[bbase + blk], srci_v.at[q], isem)
        pltpu.async_copy(dst_h.at[bbase + blk], dsti_v.at[q], isem)
        pltpu.async_copy(w_h.at[bbase + blk], wf_v.at[q], isem)

    def iwait(blk, q):
        pltpu.make_async_copy(src_h.at[bbase + blk], srci_v.at[q],
                              isem).wait()
        pltpu.make_async_copy(dst_h.at[bbase + blk], dsti_v.at[q],
                              isem).wait()
        pltpu.make_async_copy(w_h.at[bbase + blk], wf_v.at[q],
                              isem).wait()

    def gstart(p, q, off):
        for hf in range(2):
            pltpu.async_copy(h_h.at[srci_v.at[q, off, hf]],
                             rows_v.at[p, pl.ds(hf * HB, HB)],
                             gsem[2 * p + hf])

    def gwait(p, q, off):
        for hf in range(2):
            pltpu.make_async_copy(h_h.at[srci_v.at[q, off, hf]],
                                  rows_v.at[p, pl.ds(hf * HB, HB)],
                                  gsem[2 * p + hf]).wait()

    def sstart(p, q, off):
        pltpu.async_copy(rows_v.at[p], agg_sp.at[dsti_v.at[q, off]],
                         ssem[p], add=True)

    def swait(p, q, off):
        pltpu.make_async_copy(rows_v.at[p], agg_sp.at[dsti_v.at[q, off]],
                              ssem[p]).wait()

    def scale(p, q, off):
        @plsc.parallel_loop(0, EB // L)
        def _(g):
            w16 = wf_v[q, off, pl.ds(g * L, L)]
            i16 = srci_v[q, off, g // (HB // L), pl.ds((g % (HB // L)) * L, L)]
            r16 = lax.shift_right_logical(i16, 7)
            c16 = lax.bitwise_and(i16, D - 1)
            inv16 = plsc.load_gather(inv_v, [r16, c16])
            ce16 = w16 * inv16
            for ll in range(L):
                ce = ce16[ll]
                e = g * L + ll
                for kk in range(D // L):
                    fsl = pl.ds(kk * L, L)
                    rows_v[p, e, fsl] = rows_v[p, e, fsl] * ce

    istart(0, 0)
    iwait(0, 0)
    gstart(0, 0, 0)
    pltpu.make_async_copy(zeros_h.at[rsl], agg_sp.at[rsl], zsem).wait()
    plsc.subcore_barrier()
    nblk_t = jnp.int32(nblk)

    def step(j, p):
        blk = j // BLK
        off = j % BLK
        q = blk % 2
        gwait(p, q, off)

        @pl.when(j >= 1)
        def _():
            swait(1 - p, ((j - 1) // BLK) % 2, (j - 1) % BLK)

        @pl.when((off == 0) & (blk + 1 < nblk_t))
        def _():
            istart(blk + 1, 1 - q)

        @pl.when((off == BLK - 1) & (blk + 1 < nblk_t))
        def _():
            iwait(blk + 1, 1 - q)

        @pl.when(j + 1 < nb)
        def _():
            jn = j + 1
            gstart(1 - p, (jn // BLK) % 2, jn % BLK)

        scale(p, q, off)
        sstart(p, q, off)

    def body(jj, carry):
        step(2 * jj, 0)
        step(2 * jj + 1, 1)
        return carry

    lax.fori_loop(0, nb // 2, body, 0)
    swait((nb - 1) % 2, ((nb - 1) // BLK) % 2, (nb - 1) % BLK)
    plsc.subcore_barrier()

    pltpu.sync_copy(agg_sp.at[rsl], aggp_out.at[c, rsl])


def _edge_call(h, degs3, src4, dst3, w3, zeros, nb):
    return pl.kernel(
        functools.partial(_edge_body, nb),
        out_type=jax.ShapeDtypeStruct((NC, NR, D), jnp.float32),
        mesh=plsc.VectorSubcoreMesh(**_MESH),
        compiler_params=_SC_PARAMS,
        scratch_types=[
            pltpu.VMEM((2, BLK, 2, HB), jnp.int32),
            pltpu.VMEM((2, BLK, EB), jnp.int32),
            pltpu.VMEM((2, BLK, EB), jnp.float32),
            pltpu.VMEM((2, EB, D), jnp.float32),
            pltpu.VMEM((NPAD // D, D), jnp.float32),
            pltpu.SemaphoreType.DMA,
            [pltpu.SemaphoreType.DMA] * 4,
            [pltpu.SemaphoreType.DMA] * 2,
            pltpu.SemaphoreType.DMA,
            pltpu.VMEM_SHARED((NR, D), jnp.float32),
        ],
    )(h, degs3, src4, dst3, w3, zeros)


def _mm_body(x_ref, w_ref, o_ref):
    o_ref[:, :] = jnp.dot(x_ref[:, :].astype(jnp.bfloat16),
                          w_ref[:, :].astype(jnp.bfloat16),
                          preferred_element_type=jnp.float32)


def _mm_call(x, w_conv):
    return pl.pallas_call(
        _mm_body,
        out_shape=jax.ShapeDtypeStruct((N, D), jnp.float32),
    )(x, w_conv)


def _tail_body(x_ref, aggp_ref, degd_ref, bconv_ref, gamma_ref, beta_ref,
               w1_ref, b1_ref, w2_ref, b2_ref, o_ref):
    degd = degd_ref[0, :N] + degd_ref[1, :N]
    inv_dst = lax.rsqrt(jnp.maximum(degd, 1e-12))
    gamma = gamma_ref[:][None, :]
    beta = beta_ref[:][None, :]

    agg = ((aggp_ref[0, :N] + aggp_ref[1, :N]) * inv_dst[:, None]
           + bconv_ref[:][None, :])
    t = x_ref[:, :] + agg
    mu = jnp.mean(t, axis=0, keepdims=True)
    var = jnp.mean((t - mu) ** 2, axis=0, keepdims=True)
    x1 = gamma * (t - mu) * lax.rsqrt(var + 1e-5) + beta

    ffh = jnp.maximum(
        jnp.dot(x1.astype(jnp.bfloat16), w1_ref[:, :].astype(jnp.bfloat16),
                preferred_element_type=jnp.float32)
        + b1_ref[:][None, :], 0.0)
    ff = (jnp.dot(ffh.astype(jnp.bfloat16),
                  w2_ref[:, :].astype(jnp.bfloat16),
                  preferred_element_type=jnp.float32)
          + b2_ref[:][None, :])

    t2 = x1 + ff
    mu2 = jnp.mean(t2, axis=0, keepdims=True)
    var2 = jnp.mean((t2 - mu2) ** 2, axis=0, keepdims=True)
    o_ref[:, :] = gamma * (t2 - mu2) * lax.rsqrt(var2 + 1e-5) + beta


def _tail_call(x, aggp, degd, b_conv, gamma, beta, w1, b1, w2, b2):
    return pl.pallas_call(
        _tail_body,
        out_shape=jax.ShapeDtypeStruct((N, D), jnp.float32),
    )(x, aggp, degd, b_conv, gamma, beta, w1, b1, w2, b2)


def kernel(x, edge_index, edge_weight, W_conv, b_conv, gamma, beta,
           W1, b1, W2, b2):
    src = edge_index[0].astype(jnp.int32)
    dst = edge_index[1].astype(jnp.int32)
    w = edge_weight.astype(jnp.float32)
    e = src.shape[0]

    align = NW * EB * 2 * BLK
    e_pad = -(-e // align) * align
    nb = e_pad // (NW * EB)
    pad = e_pad - e

    pad_idx = jnp.arange(pad, dtype=jnp.int32) % N
    src_p = jnp.concatenate([src, pad_idx])
    dst_p = jnp.concatenate([dst, pad_idx])
    w_p = jnp.pad(w, (0, pad))
    src3 = src_p.reshape(NW * nb // BLK, BLK, EB)
    dst3 = dst_p.reshape(NW * nb // BLK, BLK, EB)
    w3 = w_p.reshape(NW * nb // BLK, BLK, EB)
    zeros = jnp.zeros((NR, D), jnp.float32)

    degs, degd = _deg_call(src_p.reshape(NW, nb, EB),
                           dst_p.reshape(NW, nb, EB),
                           w_p.reshape(NW, nb, EB), nb)
    degs3 = degs.reshape(NC, NPAD // D, D)
    degd = degd.reshape(NC, NPAD)
    h = _mm_call(x, W_conv)
    aggp = _edge_call(h, degs3, src3.reshape(-1, BLK, 2, HB), dst3, w3,
                      zeros, nb)
    return _tail_call(x, aggp, degd, b_conv, gamma, beta, W1, b1, W2, b2)
